# baseline (speedup 1.0000x reference)
import functools

import jax
import jax.numpy as jnp
from jax import lax
from jax.experimental import pallas as pl
from jax.experimental.pallas import tpu as pltpu
from jax.experimental.pallas import tpu_sc as plsc

N = 10000
E = 160000
D = 256
DH = D // 2
NC, NS, L = 2, 16, 16
EPT = E // NS
CH = 80
NCHUNK = EPT // CH


def _front_tc(x, w2):
    blk = 1000
    nb = N // blk

    def body(x_ref, w_ref, xo_ref, ab_ref):
        c = pl.program_id(1)
        xr = x_ref[...]
        xo_ref[...] = xr.astype(jnp.bfloat16)
        p = jnp.dot(xr, w_ref[...], preferred_element_type=jnp.float32)

        @pl.when(c == 0)
        def _():
            ab_ref[...] = p

        @pl.when(c == 1)
        def _():
            ab_ref[...] += p

    return pl.pallas_call(
        body,
        grid=(nb, 2),
        in_specs=[
            pl.BlockSpec((blk, DH), lambda i, c: (i, c)),
            pl.BlockSpec((DH, 2), lambda i, c: (c, 0)),
        ],
        out_specs=[
            pl.BlockSpec((blk, DH), lambda i, c: (c * nb + i, 0)),
            pl.BlockSpec((blk, 2), lambda i, c: (i, 0)),
        ],
        out_shape=[
            jax.ShapeDtypeStruct((2 * N, DH), jnp.bfloat16),
            jax.ShapeDtypeStruct((N, 2), jnp.float32),
        ],
    )(x, w2)


def _final_tc(agg2, w_lin, b2, x):
    blk = 400
    nb = N // blk
    dn = (((1,), (1,)), ((), ()))

    def body(a_ref, w_ref, b_ref, x_ref, o_ref):
        h = pl.program_id(1)
        p = lax.dot_general(a_ref[...].astype(jnp.float32), w_ref[...],
                            dn, preferred_element_type=jnp.float32)
        z = jnp.zeros((blk, DH), jnp.float32)
        xh = x_ref[...]

        @pl.when(h == 0)
        def _():
            o_ref[...] = p + b_ref[0:1, :] + jnp.concatenate([xh, z], axis=1)

        @pl.when(h == 1)
        def _():
            o_ref[...] += p + jnp.concatenate([z, xh], axis=1)

    return pl.pallas_call(
        body,
        grid=(nb, 2),
        in_specs=[
            pl.BlockSpec((blk, DH), lambda i, h: (h * nb + i, 0)),
            pl.BlockSpec((D, DH), lambda i, h: (0, h)),
            pl.BlockSpec((8, D), lambda i, h: (0, 0)),
            pl.BlockSpec((blk, DH), lambda i, h: (i, h)),
        ],
        out_specs=pl.BlockSpec((blk, D), lambda i, h: (i, 0)),
        out_shape=jax.ShapeDtypeStruct((N, D), jnp.float32),
    )(agg2, w_lin, b2, x)


def _sc_kernel(xs2, ei, ab, battn16, zrows):
    mesh = plsc.VectorSubcoreMesh(core_axis_name="c", subcore_axis_name="s")

    @functools.partial(
        pl.kernel,
        out_type=jax.ShapeDtypeStruct((2 * N, DH), jnp.bfloat16),
        mesh=mesh,
        compiler_params=pltpu.CompilerParams(use_tc_tiling_on_sc=False,
                                             needs_layout_passes=False),
        scratch_types=[
            pltpu.VMEM((EPT,), jnp.int32),
            pltpu.VMEM((NCHUNK, CH), jnp.int32),
            pltpu.VMEM((NCHUNK, CH), jnp.int32),
            pltpu.VMEM((NCHUNK, CH), jnp.float32),
            pltpu.VMEM((2 * N,), jnp.float32),
            pltpu.VMEM((CH, DH), jnp.bfloat16),
            pltpu.VMEM((CH, DH), jnp.bfloat16),
            pltpu.VMEM((CH, DH), jnp.bfloat16),
            pltpu.VMEM((CH, DH), jnp.bfloat16),
            pltpu.VMEM((CH, DH), jnp.bfloat16),
            pltpu.VMEM((L,), jnp.float32),
            pltpu.VMEM((2 * NS, L), jnp.float32),
            pltpu.VMEM_SHARED((N, DH), jnp.bfloat16),
            pltpu.VMEM_SHARED((2 * NS, L), jnp.float32),
            pltpu.SemaphoreType.DMA,
            pltpu.SemaphoreType.DMA,
            pltpu.SemaphoreType.DMA,
            pltpu.SemaphoreType.DMA,
            pltpu.SemaphoreType.DMA,
            pltpu.SemaphoreType.DMA,
            pltpu.SemaphoreType.DMA,
            pltpu.SemaphoreType.DMA,
            pltpu.SemaphoreType.DMA,
            pltpu.SemaphoreType.DMA,
        ],
    )
    def k(xs2_h, ei_h, ab_h, battn_h, zrows_h, out_h,
          eflat, rowi, coli, wbuf, abt, rows0, rows1, rows2, rows3, rows4,
          partv, redv, agg_s, red_s,
          semg0, semg1, semg2, semg3, semg4,
          sems0, sems1, sems2, sems3, sems4):
        c = lax.axis_index("c")
        s = lax.axis_index("s")
        coff = c * N
        NB = 5
        rows = (rows0, rows1, rows2, rows3, rows4)
        semg = (semg0, semg1, semg2, semg3, semg4)
        sems = (sems0, sems1, sems2, sems3, sems4)
        o16 = jnp.ones((L,), jnp.int32)

        pltpu.sync_copy(ab_h, abt)
        pltpu.sync_copy(battn_h, partv)
        bav = partv[...]

        def repack(dst):
            def rloop(g, _):
                for j in range(CH // L):
                    dst[g, pl.ds(j * L, L)] = eflat[pl.ds(g * CH + j * L, L)]
                return 0
            lax.fori_loop(0, NCHUNK, rloop, 0)

        pltpu.sync_copy(ei_h.at[pl.ds(s * EPT, EPT)], eflat)
        repack(rowi)
        pltpu.sync_copy(ei_h.at[pl.ds(E + s * EPT, EPT)], eflat)
        repack(coli)

        @pl.when(s < 10)
        def _zero():
            pltpu.sync_copy(zrows_h, agg_s.at[pl.ds(s * 1000, 1000)])

        def score_loop(g, mvec):
            for j in range(CH // L):
                r16 = rowi[g, pl.ds(j * L, L)]
                c16 = coli[g, pl.ds(j * L, L)]
                av = plsc.load_gather(abt, [r16 + r16])
                bv = plsc.load_gather(abt, [c16 + c16 + o16])
                sc = av + bv + bav
                sc = jnp.where(sc >= 0, sc, sc * jnp.float32(0.01))
                mvec = jnp.maximum(mvec, sc)
                wbuf[g, pl.ds(j * L, L)] = sc
                coli[g, pl.ds(j * L, L)] = c16 + coff
            return mvec
        mvec = lax.fori_loop(0, NCHUNK, score_loop,
                             jnp.full((L,), -jnp.inf, jnp.float32))

        for b in range(4):
            pltpu.async_copy(xs2_h.at[coli.at[b]], rows[b], semg[b])

        partv[...] = jnp.full((L,), jnp.max(mvec), jnp.float32)
        pltpu.sync_copy(partv, red_s.at[s])
        plsc.subcore_barrier()
        pltpu.sync_copy(red_s, redv)

        def rmax_loop(i, acc):
            return jnp.maximum(acc, redv[i, :])
        gmax = jnp.max(lax.fori_loop(0, NS, rmax_loop,
                                     jnp.full((L,), -jnp.inf, jnp.float32)))
        gmax16 = jnp.full((L,), gmax, jnp.float32)

        def exp_loop(g, svec):
            for j in range(CH // L):
                ev = jnp.exp(wbuf[g, pl.ds(j * L, L)] - gmax16)
                wbuf[g, pl.ds(j * L, L)] = ev
                svec = svec + ev
            return svec
        svec = lax.fori_loop(0, NCHUNK, exp_loop, jnp.zeros((L,), jnp.float32))

        partv[...] = jnp.full((L,), jnp.sum(svec), jnp.float32)
        pltpu.sync_copy(partv, red_s.at[NS + s])
        plsc.subcore_barrier()
        pltpu.sync_copy(red_s, redv)

        def rsum_loop(i, acc):
            return acc + redv[NS + i, :]
        zsum = jnp.sum(lax.fori_loop(0, NS, rsum_loop,
                                     jnp.zeros((L,), jnp.float32)))
        inv16 = jnp.full((L,), jnp.float32(1.0), jnp.float32) / jnp.full(
            (L,), zsum, jnp.float32)

        def pipe_body(t, _):
            for b in range(NB):
                gi = NB * t + b
                pltpu.make_async_copy(
                    xs2_h.at[coli.at[gi]], rows[b], semg[b]).wait()

                def scale_loop(eo, _2):
                    for kk in range(4):
                        e = eo * 4 + kk
                        w16 = plsc.load_gather(
                            wbuf, [jnp.full((L,), gi, jnp.int32),
                                   jnp.full((L,), e, jnp.int32)]) * inv16
                        w32 = plsc.pack(w16, w16,
                                        format=plsc.PackFormat.INTERLEAVED)
                        for j in range(DH // (2 * L)):
                            sl = pl.ds(j * 2 * L, 2 * L)
                            rows[b][e, sl] = rows[b][e, sl] * w32
                    return 0
                lax.fori_loop(0, CH // 4, scale_loop, 0)
                pltpu.async_copy(rows[b], agg_s.at[rowi.at[gi]], sems[b],
                                 add=True)

                bp = (b + NB - 1) % NB

                @pl.when(gi >= 1)
                def _drain():
                    pltpu.make_async_copy(
                        rows[bp], agg_s.at[rowi.at[gi - 1]],
                        sems[bp]).wait()

                @pl.when(gi + NB - 1 < NCHUNK)
                def _pf():
                    pltpu.async_copy(
                        xs2_h.at[coli.at[gi + NB - 1]], rows[bp], semg[bp])
            return 0
        lax.fori_loop(0, NCHUNK // NB, pipe_body, 0)
        pltpu.make_async_copy(rows[(NCHUNK - 1) % NB],
                              agg_s.at[rowi.at[NCHUNK - 1]],
                              sems[(NCHUNK - 1) % NB]).wait()
        plsc.subcore_barrier()

        @pl.when(s < 10)
        def _writeout():
            pltpu.sync_copy(agg_s.at[pl.ds(s * 1000, 1000)],
                            out_h.at[pl.ds(coff + s * 1000, 1000)])

    return k(xs2, ei, ab, battn16, zrows)


def kernel(x, edge_index, W_lin, b_lin, W_attn, b_attn):
    x = x.astype(jnp.float32)
    ei = edge_index.astype(jnp.int32).reshape(2 * E)

    w2 = jnp.stack([W_attn[0, :D], W_attn[0, D:]], axis=1).astype(jnp.float32)
    xs2, ab = _front_tc(x, w2)

    ab = ab.reshape(2 * N)
    battn16 = jnp.broadcast_to(b_attn.astype(jnp.float32), (L,))
    zrows = jnp.zeros((1000, DH), jnp.bfloat16)

    agg2 = _sc_kernel(xs2, ei, ab, battn16, zrows)

    b2 = jnp.broadcast_to(b_lin.astype(jnp.float32), (8, D))
    out = _final_tc(agg2, W_lin.astype(jnp.float32), b2, x)
    return out


# speedup vs baseline: 1.0771x; 1.0771x over previous
import functools

import jax
import jax.numpy as jnp
from jax import lax
from jax.experimental import pallas as pl
from jax.experimental.pallas import tpu as pltpu
from jax.experimental.pallas import tpu_sc as plsc

N = 10000
E = 160000
D = 256
DH = D // 2
NC, NS, L = 2, 16, 16
EPT = E // NS
CH = 80
NCHUNK = EPT // CH


def _front_tc(x, w2):
    blk = 1000
    nb = N // blk

    def body(x_ref, w_ref, xo_ref, ab_ref):
        c = pl.program_id(1)
        xr = x_ref[...]
        xo_ref[...] = xr.astype(jnp.bfloat16)
        p = jnp.dot(xr, w_ref[...], preferred_element_type=jnp.float32)

        @pl.when(c == 0)
        def _():
            ab_ref[...] = p

        @pl.when(c == 1)
        def _():
            ab_ref[...] += p

    return pl.pallas_call(
        body,
        grid=(nb, 2),
        in_specs=[
            pl.BlockSpec((blk, DH), lambda i, c: (i, c)),
            pl.BlockSpec((DH, 2), lambda i, c: (c, 0)),
        ],
        out_specs=[
            pl.BlockSpec((blk, DH), lambda i, c: (c * nb + i, 0)),
            pl.BlockSpec((blk, 2), lambda i, c: (i, 0)),
        ],
        out_shape=[
            jax.ShapeDtypeStruct((2 * N, DH), jnp.bfloat16),
            jax.ShapeDtypeStruct((N, 2), jnp.float32),
        ],
    )(x, w2)


def _final_tc(agg2, w_lin, b2, x):
    blk = 400
    nb = N // blk
    dn = (((1,), (1,)), ((), ()))

    def body(al_ref, ah_ref, w_ref, b_ref, x_ref, o_ref):
        acc = lax.dot_general(al_ref[...].astype(jnp.float32),
                              w_ref[:, :DH], dn,
                              preferred_element_type=jnp.float32)
        acc += lax.dot_general(ah_ref[...].astype(jnp.float32),
                               w_ref[:, DH:], dn,
                               preferred_element_type=jnp.float32)
        o_ref[...] = acc + b_ref[0:1, :] + x_ref[...]

    return pl.pallas_call(
        body,
        grid=(nb,),
        in_specs=[
            pl.BlockSpec((blk, DH), lambda i: (i, 0)),
            pl.BlockSpec((blk, DH), lambda i: (nb + i, 0)),
            pl.BlockSpec((D, D), lambda i: (0, 0)),
            pl.BlockSpec((8, D), lambda i: (0, 0)),
            pl.BlockSpec((blk, D), lambda i: (i, 0)),
        ],
        out_specs=pl.BlockSpec((blk, D), lambda i: (i, 0)),
        out_shape=jax.ShapeDtypeStruct((N, D), jnp.float32),
    )(agg2, agg2, w_lin, b2, x)


def _sc_kernel(xs2, ei, ab, battn16, zrows):
    mesh = plsc.VectorSubcoreMesh(core_axis_name="c", subcore_axis_name="s")

    @functools.partial(
        pl.kernel,
        out_type=jax.ShapeDtypeStruct((2 * N, DH), jnp.bfloat16),
        mesh=mesh,
        compiler_params=pltpu.CompilerParams(use_tc_tiling_on_sc=False,
                                             needs_layout_passes=False),
        scratch_types=[
            pltpu.VMEM((EPT,), jnp.int32),
            pltpu.VMEM((NCHUNK, CH), jnp.int32),
            pltpu.VMEM((NCHUNK, CH), jnp.int32),
            pltpu.VMEM((NCHUNK, CH), jnp.float32),
            pltpu.VMEM((2 * N,), jnp.float32),
            pltpu.VMEM((CH, DH), jnp.bfloat16),
            pltpu.VMEM((CH, DH), jnp.bfloat16),
            pltpu.VMEM((CH, DH), jnp.bfloat16),
            pltpu.VMEM((CH, DH), jnp.bfloat16),
            pltpu.VMEM((CH, DH), jnp.bfloat16),
            pltpu.VMEM((L,), jnp.float32),
            pltpu.VMEM((2 * NS, L), jnp.float32),
            pltpu.VMEM_SHARED((N, DH), jnp.bfloat16),
            pltpu.VMEM_SHARED((2 * NS, L), jnp.float32),
            pltpu.SemaphoreType.DMA,
            pltpu.SemaphoreType.DMA,
            pltpu.SemaphoreType.DMA,
            pltpu.SemaphoreType.DMA,
            pltpu.SemaphoreType.DMA,
            pltpu.SemaphoreType.DMA,
            pltpu.SemaphoreType.DMA,
            pltpu.SemaphoreType.DMA,
            pltpu.SemaphoreType.DMA,
            pltpu.SemaphoreType.DMA,
        ],
    )
    def k(xs2_h, ei_h, ab_h, battn_h, zrows_h, out_h,
          eflat, rowi, coli, wbuf, abt, rows0, rows1, rows2, rows3, rows4,
          partv, redv, agg_s, red_s,
          semg0, semg1, semg2, semg3, semg4,
          sems0, sems1, sems2, sems3, sems4):
        c = lax.axis_index("c")
        s = lax.axis_index("s")
        coff = c * N
        NB = 5
        rows = (rows0, rows1, rows2, rows3, rows4)
        semg = (semg0, semg1, semg2, semg3, semg4)
        sems = (sems0, sems1, sems2, sems3, sems4)
        o16 = jnp.ones((L,), jnp.int32)

        pltpu.sync_copy(ab_h, abt)
        pltpu.sync_copy(battn_h, partv)
        bav = partv[...]

        def repack(dst):
            def rloop(g, _):
                for j in range(CH // L):
                    dst[g, pl.ds(j * L, L)] = eflat[pl.ds(g * CH + j * L, L)]
                return 0
            lax.fori_loop(0, NCHUNK, rloop, 0)

        pltpu.sync_copy(ei_h.at[pl.ds(s * EPT, EPT)], eflat)
        repack(rowi)
        pltpu.sync_copy(ei_h.at[pl.ds(E + s * EPT, EPT)], eflat)
        repack(coli)

        @pl.when(s < 10)
        def _zero():
            pltpu.sync_copy(zrows_h, agg_s.at[pl.ds(s * 1000, 1000)])

        def score_loop(g, mvec):
            for j in range(CH // L):
                r16 = rowi[g, pl.ds(j * L, L)]
                c16 = coli[g, pl.ds(j * L, L)]
                av = plsc.load_gather(abt, [r16 + r16])
                bv = plsc.load_gather(abt, [c16 + c16 + o16])
                sc = av + bv + bav
                sc = jnp.where(sc >= 0, sc, sc * jnp.float32(0.01))
                mvec = jnp.maximum(mvec, sc)
                wbuf[g, pl.ds(j * L, L)] = sc
                coli[g, pl.ds(j * L, L)] = c16 + coff
            return mvec
        mvec = lax.fori_loop(0, NCHUNK, score_loop,
                             jnp.full((L,), -jnp.inf, jnp.float32))

        for b in range(4):
            pltpu.async_copy(xs2_h.at[coli.at[b]], rows[b], semg[b])

        partv[...] = jnp.full((L,), jnp.max(mvec), jnp.float32)
        pltpu.sync_copy(partv, red_s.at[s])
        plsc.subcore_barrier()
        pltpu.sync_copy(red_s, redv)

        def rmax_loop(i, acc):
            return jnp.maximum(acc, redv[i, :])
        gmax = jnp.max(lax.fori_loop(0, NS, rmax_loop,
                                     jnp.full((L,), -jnp.inf, jnp.float32)))
        gmax16 = jnp.full((L,), gmax, jnp.float32)

        def exp_loop(g, svec):
            for j in range(CH // L):
                ev = jnp.exp(wbuf[g, pl.ds(j * L, L)] - gmax16)
                wbuf[g, pl.ds(j * L, L)] = ev
                svec = svec + ev
            return svec
        svec = lax.fori_loop(0, NCHUNK, exp_loop, jnp.zeros((L,), jnp.float32))

        partv[...] = jnp.full((L,), jnp.sum(svec), jnp.float32)
        pltpu.sync_copy(partv, red_s.at[NS + s])
        plsc.subcore_barrier()
        pltpu.sync_copy(red_s, redv)

        def rsum_loop(i, acc):
            return acc + redv[NS + i, :]
        zsum = jnp.sum(lax.fori_loop(0, NS, rsum_loop,
                                     jnp.zeros((L,), jnp.float32)))
        inv16 = jnp.full((L,), jnp.float32(1.0), jnp.float32) / jnp.full(
            (L,), zsum, jnp.float32)

        def pipe_body(t, _):
            for b in range(NB):
                gi = NB * t + b
                pltpu.make_async_copy(
                    xs2_h.at[coli.at[gi]], rows[b], semg[b]).wait()

                def scale_loop(eo, _2):
                    for kk in range(4):
                        e = eo * 4 + kk
                        w16 = plsc.load_gather(
                            wbuf, [jnp.full((L,), gi, jnp.int32),
                                   jnp.full((L,), e, jnp.int32)]) * inv16
                        w32 = plsc.pack(w16, w16,
                                        format=plsc.PackFormat.INTERLEAVED)
                        for j in range(DH // (2 * L)):
                            sl = pl.ds(j * 2 * L, 2 * L)
                            rows[b][e, sl] = rows[b][e, sl] * w32
                    return 0
                lax.fori_loop(0, CH // 4, scale_loop, 0)
                pltpu.async_copy(rows[b], agg_s.at[rowi.at[gi]], sems[b],
                                 add=True)

                bp = (b + NB - 1) % NB

                @pl.when(gi >= 1)
                def _drain():
                    pltpu.make_async_copy(
                        rows[bp], agg_s.at[rowi.at[gi - 1]],
                        sems[bp]).wait()

                @pl.when(gi + NB - 1 < NCHUNK)
                def _pf():
                    pltpu.async_copy(
                        xs2_h.at[coli.at[gi + NB - 1]], rows[bp], semg[bp])
            return 0
        lax.fori_loop(0, NCHUNK // NB, pipe_body, 0)
        pltpu.make_async_copy(rows[(NCHUNK - 1) % NB],
                              agg_s.at[rowi.at[NCHUNK - 1]],
                              sems[(NCHUNK - 1) % NB]).wait()
        plsc.subcore_barrier()

        @pl.when(s < 10)
        def _writeout():
            pltpu.sync_copy(agg_s.at[pl.ds(s * 1000, 1000)],
                            out_h.at[pl.ds(coff + s * 1000, 1000)])

    return k(xs2, ei, ab, battn16, zrows)


def kernel(x, edge_index, W_lin, b_lin, W_attn, b_attn):
    x = x.astype(jnp.float32)
    ei = edge_index.astype(jnp.int32).reshape(2 * E)

    w2 = jnp.stack([W_attn[0, :D], W_attn[0, D:]], axis=1).astype(jnp.float32)
    xs2, ab = _front_tc(x, w2)

    ab = ab.reshape(2 * N)
    battn16 = jnp.broadcast_to(b_attn.astype(jnp.float32), (L,))
    zrows = jnp.zeros((1000, DH), jnp.bfloat16)

    agg2 = _sc_kernel(xs2, ei, ab, battn16, zrows)

    b2 = jnp.broadcast_to(b_lin.astype(jnp.float32), (8, D))
    out = _final_tc(agg2, W_lin.astype(jnp.float32), b2, x)
    return out
